# baseline (speedup 1.0000x reference)
import functools

import jax
import jax.numpy as jnp
from jax import lax
from jax.experimental import pallas as pl
from jax.experimental.pallas import tpu as pltpu
from jax.experimental.pallas import tpu_sc as plsc

NC = 2
NS = 16
NW = NC * NS
L = 16

D = 32
SCALE = float(D) ** 0.5
B = 4096
S = 200
CHUNK = 128
NBUF = 4


def _emb_body(ids_hbm, table_hbm, out_hbm, idx_v, *bufs):
    g_bufs = bufs[:NBUF]
    t_bufs = bufs[NBUF : 2 * NBUF]
    gsems = bufs[2 * NBUF : 3 * NBUF]
    ssems = bufs[3 * NBUF :]
    wid = lax.axis_index("s") * NC + lax.axis_index("c")

    pltpu.sync_copy(ids_hbm.at[:, wid], idx_v)

    for b in range(NBUF):
        pltpu.async_copy(table_hbm.at[idx_v.at[b // 8, b % 8]], g_bufs[b], gsems[b])

    iota = lax.iota(jnp.int32, L)

    @pl.loop(0, S // NBUF)
    def _group(g):
        for b in range(NBUF):
            j = g * NBUF + b
            pltpu.make_async_copy(
                table_hbm.at[idx_v.at[j // 8, j % 8]], g_bufs[b], gsems[b]
            ).wait()

            @pl.when(g >= 1)
            def _wait_store():
                pltpu.make_async_copy(
                    t_bufs[b], out_hbm.at[j - NBUF, :, wid], ssems[b]
                ).wait()

            @pl.loop(0, D)
            def _col(d):
                dcol = jnp.full((L,), d, jnp.int32)
                for j0 in range(CHUNK // L):
                    rows = iota + (j0 * L)
                    vals = plsc.load_gather(g_bufs[b], [rows, dcol])
                    t_bufs[b][d >> 3, pl.ds((d & 7) * CHUNK + j0 * L, L)] = (
                        vals * SCALE
                    )

            @pl.when(g < S // NBUF - 1)
            def _refill():
                jn = j + NBUF
                pltpu.async_copy(
                    table_hbm.at[idx_v.at[jn // 8, jn % 8]], g_bufs[b], gsems[b]
                )

            pltpu.async_copy(t_bufs[b], out_hbm.at[j, :, wid], ssems[b])

    for k in range(S - NBUF, S):
        b = k % NBUF
        pltpu.make_async_copy(
            t_bufs[b], out_hbm.at[k, :, wid], ssems[b]
        ).wait()


@jax.jit
def _emb_call(ids_t, emb_var):
    mesh = plsc.VectorSubcoreMesh(
        core_axis_name="c", subcore_axis_name="s", num_cores=NC, num_subcores=NS
    )
    fn = pl.kernel(
        _emb_body,
        out_type=jax.ShapeDtypeStruct((S, D // 8, NW, 8 * CHUNK), jnp.float32),
        mesh=mesh,
        scratch_types=[pltpu.VMEM((S // 8, 8, CHUNK), jnp.int32)]
        + [pltpu.VMEM((CHUNK, D), jnp.float32) for _ in range(NBUF)]
        + [pltpu.VMEM((D // 8, 8 * CHUNK), jnp.float32) for _ in range(NBUF)]
        + [pltpu.SemaphoreType.DMA] * (2 * NBUF),
        compiler_params=pltpu.CompilerParams(use_tc_tiling_on_sc=False, needs_layout_passes=False),
    )
    return fn(ids_t, emb_var)


def kernel(ids, emb_var):
    ids_x = (
        jnp.asarray(ids, jnp.int32)
        .T.reshape(S // 8, 8, NW, CHUNK)
        .transpose(0, 2, 1, 3)
    )
    out_phys = _emb_call(ids_x, emb_var)
    return (
        out_phys.reshape(S, D // 8, NW, 8, CHUNK)
        .transpose(2, 4, 0, 1, 3)
        .reshape(B, S, D)
    )


# speedup vs baseline: 1.8441x; 1.8441x over previous
import jax
import jax.numpy as jnp
from jax import lax
from jax.experimental import pallas as pl
from jax.experimental.pallas import tpu as pltpu
from jax.experimental.pallas import tpu_sc as plsc

NC = 2
NS = 16
NW = NC * NS
L = 16

D = 32
SCALE = float(D) ** 0.5
B = 4096
S = 200
V = 1_000_000
CHUNK = 128
NB1 = 2
NBUF = 4

CT_FULL = V // 128
CT_RING = 244
MAGIC = 0x5CAB51AB
POLL_LIMIT = 200000


def _transpose_tile(src, dst, dtab, iota, n_rows, scale=None):
    @pl.loop(0, D)
    def _d0(d0):
        dvec = dtab[d0, 0:L]
        for r0 in range(0, n_rows, L):
            rvec = iota + r0
            vals = plsc.load_gather(src, [dvec, rvec])
            if scale is not None:
                vals = vals * scale
            plsc.store_scatter(dst, [rvec, dvec], vals)


def _emb_body(ids_hbm, tab_t_hbm, tail_hbm, out_hbm, rmaj, flags,
              dtab, fbuf, *bufs):
    o = 0
    blk = bufs[o:o + NB1]; o += NB1
    tr = bufs[o:o + NB1]; o += NB1
    idx_b = bufs[o:o + NBUF]; o += NBUF
    g_bufs = bufs[o:o + NBUF]; o += NBUF
    t_bufs = bufs[o:o + NBUF]; o += NBUF
    isems = bufs[o:o + NB1]; o += NB1
    osems = bufs[o:o + NB1]; o += NB1
    xsems = bufs[o:o + NBUF]; o += NBUF
    gsems = bufs[o:o + NBUF]; o += NBUF
    ssems = bufs[o:o + NBUF]

    cid = lax.axis_index("c")
    sid = lax.axis_index("s")
    wid = sid * NC + cid
    iota = lax.iota(jnp.int32, L)

    @pl.when(sid == 0)
    def _reset_flag():
        fbuf[0:L] = jnp.zeros((L,), jnp.int32)
        pltpu.sync_copy(fbuf, flags.at[cid])

    @pl.loop(0, D)
    def _mk_tabs(d0):
        dtab[d0, 0:L] = (d0 + iota) & (D - 1)

    for b in range(NB1):
        ct = wid + 32 * b
        pltpu.async_copy(
            tab_t_hbm.at[:, pl.ds(ct * 128, 128)], blk[b], isems[b]
        )

    @pl.loop(0, CT_RING // NB1)
    def _p1(g):
        for b in range(NB1):
            k = g * NB1 + b
            ct = wid + 32 * k
            pltpu.make_async_copy(
                tab_t_hbm.at[:, pl.ds(ct * 128, 128)], blk[b], isems[b]
            ).wait()

            @pl.when(g >= 1)
            def _wait_store():
                ctp = ct - 32 * NB1
                pltpu.make_async_copy(
                    tr[b], rmaj.at[pl.ds(ctp * 128, 128), :], osems[b]
                ).wait()

            _transpose_tile(blk[b], tr[b], dtab, iota, 128)

            @pl.when(g < CT_RING // NB1 - 1)
            def _refill():
                ctn = ct + 32 * NB1
                pltpu.async_copy(
                    tab_t_hbm.at[:, pl.ds(ctn * 128, 128)], blk[b], isems[b]
                )

            pltpu.async_copy(
                tr[b], rmaj.at[pl.ds(ct * 128, 128), :], osems[b]
            )

    for b in range(NB1):
        ct = wid + 32 * (CT_RING - NB1 + b)
        pltpu.make_async_copy(
            tr[b], rmaj.at[pl.ds(ct * 128, 128), :], osems[b]
        ).wait()

    ct_t = wid + 32 * CT_RING

    @pl.when(ct_t < CT_FULL)
    def _tail_full():
        pltpu.sync_copy(tab_t_hbm.at[:, pl.ds(ct_t * 128, 128)], blk[0])
        _transpose_tile(blk[0], tr[0], dtab, iota, 128)
        pltpu.sync_copy(tr[0], rmaj.at[pl.ds(ct_t * 128, 128), :])

    @pl.when(ct_t == CT_FULL)
    def _tail_rows():
        nrem = V - CT_FULL * 128
        pltpu.sync_copy(tail_hbm, blk[0].at[pl.ds(0, L), :])
        for t in range(nrem * D // L):
            vals = blk[0][t // 8, (t % 8) * L : (t % 8 + 1) * L]
            tr[0][t // 2, (t % 2) * L : (t % 2 + 1) * L] = vals
        pltpu.sync_copy(
            tr[0].at[pl.ds(0, nrem), :], rmaj.at[pl.ds(CT_FULL * 128, nrem), :]
        )

    plsc.subcore_barrier()

    @pl.when(sid == 0)
    def _handshake():
        fbuf[0:L] = jnp.full((L,), MAGIC, jnp.int32)
        pltpu.sync_copy(fbuf, flags.at[cid])

        def _cond(st):
            i, done = st
            return jnp.logical_and(i < POLL_LIMIT, jnp.logical_not(done))

        def _poll(st):
            i, _ = st
            pltpu.sync_copy(flags.at[1 - cid], fbuf)
            v = fbuf[0:L]
            nmatch = jnp.sum((v == MAGIC).astype(jnp.int32))
            return i + 1, nmatch == L

        lax.while_loop(_cond, _poll, (jnp.int32(0), False))

    plsc.subcore_barrier()

    for b in range(NBUF):
        pltpu.sync_copy(ids_hbm.at[b // 8, wid, b % 8], idx_b[b].at[0])
        pltpu.async_copy(rmaj.at[idx_b[b].at[0]], g_bufs[b], gsems[b])

    @pl.loop(0, S // NBUF)
    def _p2(g):
        for b in range(NBUF):
            j = g * NBUF + b
            pltpu.make_async_copy(
                rmaj.at[idx_b[b].at[0]], g_bufs[b], gsems[b]
            ).wait()

            @pl.when(g >= 1)
            def _wait_store():
                pltpu.make_async_copy(
                    t_bufs[b], out_hbm.at[j - NBUF, :, wid], ssems[b]
                ).wait()

            @pl.when(g < S // NBUF - 1)
            def _idx_prefetch():
                jn = j + NBUF
                pltpu.async_copy(
                    ids_hbm.at[jn // 8, wid, jn % 8], idx_b[b].at[0], xsems[b]
                )

            @pl.loop(0, D)
            def _d0(d0):
                dvec = dtab[d0, 0:L]
                dtvec = dvec >> 3
                drvec = dvec & 7
                for r0 in range(0, CHUNK, L):
                    rvec = iota + r0
                    vals = plsc.load_gather(g_bufs[b], [rvec, dvec])
                    plsc.store_scatter(
                        t_bufs[b], [dtvec, drvec, rvec], vals * SCALE
                    )

            @pl.when(g < S // NBUF - 1)
            def _refill():
                jn = j + NBUF
                pltpu.make_async_copy(
                    ids_hbm.at[jn // 8, wid, jn % 8], idx_b[b].at[0], xsems[b]
                ).wait()
                pltpu.async_copy(rmaj.at[idx_b[b].at[0]], g_bufs[b], gsems[b])

            pltpu.async_copy(t_bufs[b], out_hbm.at[j, :, wid], ssems[b])

    for k in range(S - NBUF, S):
        b = k % NBUF
        pltpu.make_async_copy(
            t_bufs[b], out_hbm.at[k, :, wid], ssems[b]
        ).wait()


@jax.jit
def _emb_call(ids_x, tab_t, tail128):
    mesh = plsc.VectorSubcoreMesh(
        core_axis_name="c", subcore_axis_name="s", num_cores=NC, num_subcores=NS
    )
    fn = pl.kernel(
        _emb_body,
        out_type=jax.ShapeDtypeStruct((S, D // 8, NW, 8, CHUNK), jnp.float32),
        mesh=mesh,
        scratch_types=[
            pltpu.HBM((V, D), jnp.float32),
            pltpu.HBM((NC, L), jnp.int32),
            pltpu.VMEM((D, L), jnp.int32),
            pltpu.VMEM((L,), jnp.int32),
        ]
        + [pltpu.VMEM((D, CHUNK), jnp.float32) for _ in range(NB1)]
        + [pltpu.VMEM((CHUNK, D), jnp.float32) for _ in range(NB1)]
        + [pltpu.VMEM((1, CHUNK), jnp.int32) for _ in range(NBUF)]
        + [pltpu.VMEM((CHUNK, D), jnp.float32) for _ in range(NBUF)]
        + [pltpu.VMEM((D // 8, 8, CHUNK), jnp.float32) for _ in range(NBUF)]
        + [pltpu.SemaphoreType.DMA] * (2 * NB1 + 3 * NBUF),
        compiler_params=pltpu.CompilerParams(
            use_tc_tiling_on_sc=True, needs_layout_passes=False
        ),
    )
    return fn(ids_x, tab_t, tail128)


def kernel(ids, emb_var):
    ids_x = (
        jnp.asarray(ids, jnp.int32)
        .T.reshape(S // 8, 8, NW, CHUNK)
        .transpose(0, 2, 1, 3)
    )
    tab_t = emb_var.T
    tail128 = emb_var[CT_FULL * 128 :, :].reshape(L, CHUNK)
    out_phys = _emb_call(ids_x, tab_t, tail128)
    return (
        out_phys.transpose(2, 4, 0, 1, 3)
        .reshape(B, S, D)
    )


# speedup vs baseline: 3.0418x; 1.6495x over previous
import jax
import jax.numpy as jnp
from jax import lax
from jax.experimental import pallas as pl
from jax.experimental.pallas import tpu as pltpu
from jax.experimental.pallas import tpu_sc as plsc

NC = 2
NS = 16
NW = NC * NS
L = 16

D = 32
SCALE = float(D) ** 0.5
B = 4096
S = 200
V = 1_000_000
CHUNK = 128
NB1 = 2
NBUF = 4

CT_FULL = V // 128
CT_RING = 244
MAGIC = 0x5CAB51AB
POLL_LIMIT = 200000


def _transpose_tile(src, dst, dtab, iota, n_rows, scale=None):
    @plsc.parallel_loop(0, D, unroll=4)
    def _d0(d0):
        dvec = dtab[d0, 0:L]
        for r0 in range(0, n_rows, L):
            rvec = iota + r0
            vals = plsc.load_gather(src, [dvec, rvec])
            if scale is not None:
                vals = vals * scale
            plsc.store_scatter(dst, [rvec, dvec], vals)


def _emb_body(ids_hbm, tab_t_hbm, tail_hbm, out_hbm, rmaj, flags,
              dtab, fbuf, *bufs):
    o = 0
    blk = bufs[o:o + NB1]; o += NB1
    tr = bufs[o:o + NB1]; o += NB1
    idx_b = bufs[o:o + NBUF]; o += NBUF
    g_bufs = bufs[o:o + NBUF]; o += NBUF
    t_bufs = bufs[o:o + NBUF]; o += NBUF
    isems = bufs[o:o + NB1]; o += NB1
    osems = bufs[o:o + NB1]; o += NB1
    xsems = bufs[o:o + NBUF]; o += NBUF
    gsems = bufs[o:o + NBUF]; o += NBUF
    ssems = bufs[o:o + NBUF]

    cid = lax.axis_index("c")
    sid = lax.axis_index("s")
    wid = sid * NC + cid
    iota = lax.iota(jnp.int32, L)

    @pl.when(sid == 0)
    def _reset_flag():
        fbuf[0:L] = jnp.zeros((L,), jnp.int32)
        pltpu.sync_copy(fbuf, flags.at[cid])

    @pl.loop(0, D)
    def _mk_tabs(d0):
        dtab[d0, 0:L] = (d0 + iota) & (D - 1)

    for b in range(NB1):
        ct = wid + 32 * b
        pltpu.async_copy(
            tab_t_hbm.at[:, pl.ds(ct * 128, 128)], blk[b], isems[b]
        )

    @pl.loop(0, CT_RING // NB1)
    def _p1(g):
        for b in range(NB1):
            k = g * NB1 + b
            ct = wid + 32 * k
            pltpu.make_async_copy(
                tab_t_hbm.at[:, pl.ds(ct * 128, 128)], blk[b], isems[b]
            ).wait()

            @pl.when(g >= 1)
            def _wait_store():
                ctp = ct - 32 * NB1
                pltpu.make_async_copy(
                    tr[b], rmaj.at[pl.ds(ctp * 128, 128), :], osems[b]
                ).wait()

            _transpose_tile(blk[b], tr[b], dtab, iota, 128)

            @pl.when(g < CT_RING // NB1 - 1)
            def _refill():
                ctn = ct + 32 * NB1
                pltpu.async_copy(
                    tab_t_hbm.at[:, pl.ds(ctn * 128, 128)], blk[b], isems[b]
                )

            pltpu.async_copy(
                tr[b], rmaj.at[pl.ds(ct * 128, 128), :], osems[b]
            )

    for b in range(NB1):
        ct = wid + 32 * (CT_RING - NB1 + b)
        pltpu.make_async_copy(
            tr[b], rmaj.at[pl.ds(ct * 128, 128), :], osems[b]
        ).wait()

    ct_t = wid + 32 * CT_RING

    @pl.when(ct_t < CT_FULL)
    def _tail_full():
        pltpu.sync_copy(tab_t_hbm.at[:, pl.ds(ct_t * 128, 128)], blk[0])
        _transpose_tile(blk[0], tr[0], dtab, iota, 128)
        pltpu.sync_copy(tr[0], rmaj.at[pl.ds(ct_t * 128, 128), :])

    @pl.when(ct_t == CT_FULL)
    def _tail_rows():
        nrem = V - CT_FULL * 128
        pltpu.sync_copy(tail_hbm, blk[0].at[pl.ds(0, L), :])
        for t in range(nrem * D // L):
            vals = blk[0][t // 8, (t % 8) * L : (t % 8 + 1) * L]
            tr[0][t // 2, (t % 2) * L : (t % 2 + 1) * L] = vals
        pltpu.sync_copy(
            tr[0].at[pl.ds(0, nrem), :], rmaj.at[pl.ds(CT_FULL * 128, nrem), :]
        )

    plsc.subcore_barrier()

    @pl.when(sid == 0)
    def _handshake():
        fbuf[0:L] = jnp.full((L,), MAGIC, jnp.int32)
        pltpu.sync_copy(fbuf, flags.at[cid])

        def _cond(st):
            i, done = st
            return jnp.logical_and(i < POLL_LIMIT, jnp.logical_not(done))

        def _poll(st):
            i, _ = st
            pltpu.sync_copy(flags.at[1 - cid], fbuf)
            v = fbuf[0:L]
            nmatch = jnp.sum((v == MAGIC).astype(jnp.int32))
            return i + 1, nmatch == L

        lax.while_loop(_cond, _poll, (jnp.int32(0), False))

    plsc.subcore_barrier()

    for b in range(NBUF):
        pltpu.sync_copy(ids_hbm.at[b // 8, wid, b % 8], idx_b[b].at[0])
        pltpu.async_copy(rmaj.at[idx_b[b].at[0]], g_bufs[b], gsems[b])

    @pl.loop(0, S // NBUF)
    def _p2(g):
        for b in range(NBUF):
            j = g * NBUF + b
            pltpu.make_async_copy(
                rmaj.at[idx_b[b].at[0]], g_bufs[b], gsems[b]
            ).wait()

            @pl.when(g >= 1)
            def _wait_store():
                pltpu.make_async_copy(
                    t_bufs[b], out_hbm.at[j - NBUF, :, wid], ssems[b]
                ).wait()

            @pl.when(g < S // NBUF - 1)
            def _idx_prefetch():
                jn = j + NBUF
                pltpu.async_copy(
                    ids_hbm.at[jn // 8, wid, jn % 8], idx_b[b].at[0], xsems[b]
                )

            @plsc.parallel_loop(0, D, unroll=4)
            def _d0(d0):
                dvec = dtab[d0, 0:L]
                dtvec = dvec >> 3
                drvec = dvec & 7
                for r0 in range(0, CHUNK, L):
                    rvec = iota + r0
                    vals = plsc.load_gather(g_bufs[b], [rvec, dvec])
                    plsc.store_scatter(
                        t_bufs[b], [dtvec, drvec, rvec], vals * SCALE
                    )

            @pl.when(g < S // NBUF - 1)
            def _refill():
                jn = j + NBUF
                pltpu.make_async_copy(
                    ids_hbm.at[jn // 8, wid, jn % 8], idx_b[b].at[0], xsems[b]
                ).wait()
                pltpu.async_copy(rmaj.at[idx_b[b].at[0]], g_bufs[b], gsems[b])

            pltpu.async_copy(t_bufs[b], out_hbm.at[j, :, wid], ssems[b])

    for k in range(S - NBUF, S):
        b = k % NBUF
        pltpu.make_async_copy(
            t_bufs[b], out_hbm.at[k, :, wid], ssems[b]
        ).wait()


@jax.jit
def _emb_call(ids_x, tab_t, tail128):
    mesh = plsc.VectorSubcoreMesh(
        core_axis_name="c", subcore_axis_name="s", num_cores=NC, num_subcores=NS
    )
    fn = pl.kernel(
        _emb_body,
        out_type=jax.ShapeDtypeStruct((S, D // 8, NW, 8, CHUNK), jnp.float32),
        mesh=mesh,
        scratch_types=[
            pltpu.HBM((V, D), jnp.float32),
            pltpu.HBM((NC, L), jnp.int32),
            pltpu.VMEM((D, L), jnp.int32),
            pltpu.VMEM((L,), jnp.int32),
        ]
        + [pltpu.VMEM((D, CHUNK), jnp.float32) for _ in range(NB1)]
        + [pltpu.VMEM((CHUNK, D), jnp.float32) for _ in range(NB1)]
        + [pltpu.VMEM((1, CHUNK), jnp.int32) for _ in range(NBUF)]
        + [pltpu.VMEM((CHUNK, D), jnp.float32) for _ in range(NBUF)]
        + [pltpu.VMEM((D // 8, 8, CHUNK), jnp.float32) for _ in range(NBUF)]
        + [pltpu.SemaphoreType.DMA] * (2 * NB1 + 3 * NBUF),
        compiler_params=pltpu.CompilerParams(
            use_tc_tiling_on_sc=True, needs_layout_passes=False
        ),
    )
    return fn(ids_x, tab_t, tail128)


def kernel(ids, emb_var):
    ids_x = (
        jnp.asarray(ids, jnp.int32)
        .T.reshape(S // 8, 8, NW, CHUNK)
        .transpose(0, 2, 1, 3)
    )
    tab_t = emb_var.T
    tail128 = emb_var[CT_FULL * 128 :, :].reshape(L, CHUNK)
    out_phys = _emb_call(ids_x, tab_t, tail128)
    return (
        out_phys.transpose(2, 4, 0, 1, 3)
        .reshape(B, S, D)
    )
